# Initial kernel scaffold; baseline (speedup 1.0000x reference)
import jax
import jax.numpy as jnp
from jax.experimental import pallas as pl


def kernel(inp, tokens, W1, b1, W2, b2, W3, b3):
    raise NotImplementedError("write your pallas kernel here")


# speedup vs baseline: 2.9446x; 2.9446x over previous
import functools

import jax
import jax.numpy as jnp
from jax import lax
from jax.experimental import pallas as pl
from jax.experimental.pallas import tpu as pltpu
from jax.experimental.pallas import tpu_sc as plsc

B = 128
V = 100000
L = 16
NC = 2
NS = 16
NW = NC * NS
ROWS_PER_W = B // NW
NVEC = V // L
VPW = 50
NWIN = NVEC // VPW
NEG_INF = float("-inf")


def _sc_body(inp_hbm, tok_hbm, out_hbm, row_v, tok_v, stage_v):
    c = lax.axis_index("c")
    s = lax.axis_index("s")
    wid = s * NC + c
    base_row = wid * ROWS_PER_W
    tok_base = (wid // 4) * 16
    pltpu.sync_copy(tok_hbm.at[pl.ds(tok_base, 16)], tok_v)
    iota = lax.iota(jnp.int32, L)

    for r in range(ROWS_PER_W):
        row = base_row + r
        pltpu.sync_copy(inp_hbm.at[row], row_v)

        def win_body(g, carry):
            Rk, Rv = carry
            base = pl.multiple_of(g * (VPW * L), L)
            m = row_v[pl.ds(base, L)]
            for cc in range(1, VPW):
                m = jnp.maximum(m, row_v[pl.ds(base + cc * L, L)])
            vals = g * L + iota
            sk, sv = plsc.sort_key_val(m, vals, descending=False)
            take = Rk >= sk
            mk = jnp.where(take, Rk, sk)
            mv = jnp.where(take, Rv, sv)
            return tuple(plsc.sort_key_val(mk, mv, descending=True))

        Rk0 = jnp.full((L,), NEG_INF, jnp.float32)
        Rv0 = jnp.zeros((L,), jnp.int32)
        _, Rv = lax.fori_loop(0, NWIN, win_body, (Rk0, Rv0))

        win_id = lax.shift_right_logical(Rv, 4)
        lane = jnp.bitwise_and(Rv, L - 1)
        gbase = win_id * (VPW * L) + lane

        def c_body(cc, RT):
            gath = plsc.load_gather(row_v, [gbase + cc * L])
            merged = jnp.maximum(RT, jnp.sort(gath))
            return lax.rev(jnp.sort(merged), (0,))

        RT = lax.fori_loop(0, VPW, c_body,
                           jnp.full((L,), NEG_INF, jnp.float32))

        tok_splat = plsc.load_gather(
            tok_v, [jnp.full((L,), (wid % 4) * 4 + r, jnp.int32)])
        gval = plsc.load_gather(row_v, [tok_splat])
        stage_v[...] = jnp.where(iota == L - 1, gval, RT)
        pltpu.sync_copy(stage_v, out_hbm.at[row])


@functools.cache
def _sc_topk():
    return functools.partial(
        pl.kernel,
        mesh=plsc.VectorSubcoreMesh(core_axis_name="c", subcore_axis_name="s"),
        compiler_params=pltpu.CompilerParams(needs_layout_passes=False),
        out_type=jax.ShapeDtypeStruct((B, L), jnp.float32),
        scratch_types=[
            pltpu.VMEM((V,), jnp.float32),
            pltpu.VMEM((16,), jnp.int32),
            pltpu.VMEM((L,), jnp.float32),
        ],
    )(_sc_body)


CW = 8192
NCH = -(-V // CW)


def _tc_body(tg_ref, inp_ref, w1_ref, w2_ref, w3_ref, b_ref,
             out_ref, it_ref, m_ref, acc_ref):
    j = pl.program_id(0)

    @pl.when(j == 0)
    def _():
        tg = tg_ref[...]
        col16 = lax.broadcasted_iota(jnp.int32, (B, L), 1)
        t10 = jnp.where(col16 < 10, tg, 0.0)
        dn = (((1,), (0,)), ((), ()))
        h = lax.dot_general(t10, w1_ref[...], dn,
                            preferred_element_type=jnp.float32)
        h = jnp.maximum(h + b_ref[0:1, :], 0.0)
        h = lax.dot_general(h, w2_ref[...], dn,
                            preferred_element_type=jnp.float32)
        h = jnp.maximum(h + b_ref[1:2, :], 0.0)
        h = lax.dot_general(h, w3_ref[...], dn,
                            preferred_element_type=jnp.float32)
        h = jnp.abs(h + b_ref[2:3, :])
        temp = jnp.clip(h[:, 0:1], 1e-8, 1e8)
        it_ref[...] = 1.0 / temp
        m_ref[...] = tg[:, 0:1]
        acc_ref[...] = jnp.zeros_like(acc_ref)

    x = inp_ref[...]
    cols = j * CW + lax.broadcasted_iota(jnp.int32, (B, CW), 1)
    e = jnp.exp((x - m_ref[...]) * it_ref[...])
    e = jnp.where(cols < V, e, 0.0)
    acc_ref[...] = acc_ref[...] + jnp.sum(e, axis=1, keepdims=True)

    @pl.when(j == NCH - 1)
    def _():
        g = tg_ref[...][:, L - 1:L]
        out_ref[...] = jnp.exp((g - m_ref[...]) * it_ref[...]) / acc_ref[...]


_tc_softmax = pl.pallas_call(
    _tc_body,
    grid=(NCH,),
    in_specs=[
        pl.BlockSpec((B, L), lambda j: (0, 0)),
        pl.BlockSpec((B, CW), lambda j: (0, j)),
        pl.BlockSpec((L, 128), lambda j: (0, 0)),
        pl.BlockSpec((128, 128), lambda j: (0, 0)),
        pl.BlockSpec((128, 128), lambda j: (0, 0)),
        pl.BlockSpec((8, 128), lambda j: (0, 0)),
    ],
    out_specs=pl.BlockSpec((B, 1), lambda j: (0, 0)),
    out_shape=jax.ShapeDtypeStruct((B, 1), jnp.float32),
    scratch_shapes=[
        pltpu.VMEM((B, 1), jnp.float32),
        pltpu.VMEM((B, 1), jnp.float32),
        pltpu.VMEM((B, 1), jnp.float32),
    ],
    compiler_params=pltpu.CompilerParams(
        dimension_semantics=("arbitrary",)),
)


def kernel(inp, tokens, W1, b1, W2, b2, W3, b3):
    tokens = tokens.astype(jnp.int32)
    tg = _sc_topk()(inp, tokens)
    w1p = jnp.zeros((L, 128), jnp.float32).at[:10, :5].set(W1.T)
    w2p = jnp.zeros((128, 128), jnp.float32).at[:5, :5].set(W2.T)
    w3p = jnp.zeros((128, 128), jnp.float32).at[:5, :1].set(W3.T)
    bp = (jnp.zeros((8, 128), jnp.float32)
          .at[0, :5].set(b1).at[1, :5].set(b2).at[2, 0].set(b3[0]))
    out2 = _tc_softmax(tg, inp, w1p, w2p, w3p, bp)
    return out2[:, 0]
